# baseline (speedup 1.0000x reference)
import functools

import jax
import jax.numpy as jnp
from jax import lax
from jax.experimental import pallas as pl
from jax.experimental.pallas import tpu as pltpu
from jax.experimental.pallas import tpu_sc as plsc

N_EXPERTS = 64
K = 8
BLK = 512
NC, NS, L = 2, 16, 16
NW = NC * NS


def _matmul_body(x_ref, w_ref, b_ref, s_ref):
    s_ref[...] = jax.lax.dot_general(
        x_ref[...], w_ref[...],
        dimension_numbers=(((1,), (1,)), ((), ())),
        preferred_element_type=jnp.float32,
        precision=jax.lax.Precision.DEFAULT,
    ) + b_ref[...]


def _tc_scores(x, weight, bias2d):
    grid = (x.shape[0] // BLK,)
    return pl.pallas_call(
        _matmul_body,
        grid=grid,
        in_specs=[
            pl.BlockSpec((BLK, x.shape[1]), lambda i: (i, 0)),
            pl.BlockSpec((N_EXPERTS, x.shape[1]), lambda i: (0, 0)),
            pl.BlockSpec((1, N_EXPERTS), lambda i: (0, 0)),
        ],
        out_specs=pl.BlockSpec((BLK, N_EXPERTS), lambda i: (i, 0)),
        out_shape=jax.ShapeDtypeStruct((x.shape[0], N_EXPERTS), jnp.float32),
    )(x, weight, bias2d)


def _merge_top8(ak, av, bk, bv):
    lane = lax.broadcasted_iota(jnp.int32, (L,), 0)
    sel = lane < 8
    ck = jnp.where(sel, ak, lax.rev(bk, (0,)))
    cv = jnp.where(sel, av, lax.rev(bv, (0,)))
    return plsc.sort_key_val(ck, cv, descending=True)


N_Q = 8


def _sc_gate_body(rpw, s_hbm, wout_hbm, iout_hbm, s_v, w_v, i_v,
                  sem_in, sem_out):
    wid = lax.axis_index("s") * NC + lax.axis_index("c")
    base = wid * rpw
    rq = rpw // N_Q

    in_copies = [
        pltpu.async_copy(s_hbm.at[pl.ds(base + q * rq, rq)],
                         s_v.at[pl.ds(q * rq, rq)], sem_in[q])
        for q in range(N_Q)
    ]

    lane = lax.broadcasted_iota(jnp.int32, (L,), 0)
    sel8 = lane < 8

    def _do_rows(lo, hi):
        @plsc.parallel_loop(lo, hi, step=1, unroll=2)
        def _row(r):
            sk, sv = [], []
            for c in range(N_EXPERTS // L):
                keys = s_v[r, pl.ds(c * L, L)]
                gids = lane + (c * L)
                k_s, v_s = plsc.sort_key_val(keys, gids, descending=True)
                sk.append(k_s)
                sv.append(v_s)
            mk0, mv0 = _merge_top8(sk[0], sv[0], sk[1], sv[1])
            mk1, mv1 = _merge_top8(sk[2], sv[2], sk[3], sv[3])
            fk, fv = _merge_top8(mk0, mv0, mk1, mv1)

            kmax = lax.reduce_max(fk, (0,))
            e = jnp.where(sel8, jnp.exp(fk - kmax), 0.0)
            w = e / lax.reduce_sum(e, (0,))

            rsplat = jnp.full((L,), r, jnp.int32)
            plsc.store_scatter(w_v, [rsplat, lane], w, mask=sel8)
            plsc.store_scatter(i_v, [rsplat, lane], fv, mask=sel8)

    out_copies = []
    for q in range(N_Q):
        in_copies[q].wait()
        _do_rows(q * rq, (q + 1) * rq)
        out_copies.append(pltpu.async_copy(
            w_v.at[pl.ds(q * rq, rq)],
            wout_hbm.at[pl.ds(base + q * rq, rq)], sem_out))
        out_copies.append(pltpu.async_copy(
            i_v.at[pl.ds(q * rq, rq)],
            iout_hbm.at[pl.ds(base + q * rq, rq)], sem_out))
    for h in out_copies:
        h.wait()


@functools.lru_cache(maxsize=None)
def _make_sc_gate(n_rows):
    rpw = n_rows // NW
    return pl.kernel(
        functools.partial(_sc_gate_body, rpw),
        out_type=[
            jax.ShapeDtypeStruct((n_rows, K), jnp.float32),
            jax.ShapeDtypeStruct((n_rows, K), jnp.int32),
        ],
        mesh=plsc.VectorSubcoreMesh(core_axis_name="c", subcore_axis_name="s"),
        compiler_params=pltpu.CompilerParams(needs_layout_passes=False),
        scratch_types=[
            pltpu.VMEM((rpw, N_EXPERTS), jnp.float32),
            pltpu.VMEM((rpw, K), jnp.float32),
            pltpu.VMEM((rpw, K), jnp.int32),
            [pltpu.SemaphoreType.DMA] * N_Q,
            pltpu.SemaphoreType.DMA,
        ],
    )


@jax.jit
def kernel(x, weight, bias):
    scores = _tc_scores(x, weight, bias.reshape(1, N_EXPERTS))
    return tuple(_make_sc_gate(x.shape[0])(scores))


# speedup vs baseline: 1.0225x; 1.0225x over previous
import functools

import jax
import jax.numpy as jnp
from jax import lax
from jax.experimental import pallas as pl
from jax.experimental.pallas import tpu as pltpu
from jax.experimental.pallas import tpu_sc as plsc

N_EXPERTS = 64
K = 8
BLK = 512
NC, NS, L = 2, 16, 16
NW = NC * NS


def _matmul_body(x_ref, w_ref, b_ref, s_ref):
    s_ref[...] = jax.lax.dot_general(
        x_ref[...], w_ref[...],
        dimension_numbers=(((1,), (1,)), ((), ())),
        preferred_element_type=jnp.float32,
        precision=jax.lax.Precision.DEFAULT,
    ) + b_ref[...]


def _tc_scores(x, weight, bias2d):
    grid = (x.shape[0] // BLK,)
    return pl.pallas_call(
        _matmul_body,
        grid=grid,
        in_specs=[
            pl.BlockSpec((BLK, x.shape[1]), lambda i: (i, 0)),
            pl.BlockSpec((N_EXPERTS, x.shape[1]), lambda i: (0, 0)),
            pl.BlockSpec((1, N_EXPERTS), lambda i: (0, 0)),
        ],
        out_specs=pl.BlockSpec((BLK, N_EXPERTS), lambda i: (i, 0)),
        out_shape=jax.ShapeDtypeStruct((x.shape[0], N_EXPERTS), jnp.float32),
    )(x, weight, bias2d)


def _merge_top8(ak, av, bk, bv):
    lane = lax.broadcasted_iota(jnp.int32, (L,), 0)
    sel = lane < 8
    ck = jnp.where(sel, ak, lax.rev(bk, (0,)))
    cv = jnp.where(sel, av, lax.rev(bv, (0,)))
    return plsc.sort_key_val(ck, cv, descending=True)


N_Q = 2


def _sc_gate_body(rpw, s_hbm, wout_hbm, iout_hbm, s_v, w_v, i_v,
                  sem_in, sem_out):
    wid = lax.axis_index("s") * NC + lax.axis_index("c")
    base = wid * rpw
    rq = rpw // N_Q

    in_copies = [
        pltpu.async_copy(s_hbm.at[pl.ds(base + q * rq, rq)],
                         s_v.at[pl.ds(q * rq, rq)], sem_in[q])
        for q in range(N_Q)
    ]

    lane = lax.broadcasted_iota(jnp.int32, (L,), 0)
    sel8 = lane < 8

    def _do_rows(lo, hi):
        @plsc.parallel_loop(lo, hi, step=1, unroll=2)
        def _row(r):
            sk, sv = [], []
            for c in range(N_EXPERTS // L):
                keys = s_v[r, pl.ds(c * L, L)]
                gids = lane + (c * L)
                k_s, v_s = plsc.sort_key_val(keys, gids, descending=True)
                sk.append(k_s)
                sv.append(v_s)
            mk0, mv0 = _merge_top8(sk[0], sv[0], sk[1], sv[1])
            mk1, mv1 = _merge_top8(sk[2], sv[2], sk[3], sv[3])
            fk, fv = _merge_top8(mk0, mv0, mk1, mv1)

            kmax = lax.reduce_max(fk, (0,))
            e = jnp.where(sel8, jnp.exp(fk - kmax), 0.0)
            w = e / lax.reduce_sum(e, (0,))

            rsplat = jnp.full((L,), r, jnp.int32)
            plsc.store_scatter(w_v, [rsplat, lane], w, mask=sel8)
            plsc.store_scatter(i_v, [rsplat, lane], fv, mask=sel8)

    out_copies = []
    for q in range(N_Q):
        in_copies[q].wait()
        _do_rows(q * rq, (q + 1) * rq)
        out_copies.append(pltpu.async_copy(
            w_v.at[pl.ds(q * rq, rq)],
            wout_hbm.at[pl.ds(base + q * rq, rq)], sem_out))
        out_copies.append(pltpu.async_copy(
            i_v.at[pl.ds(q * rq, rq)],
            iout_hbm.at[pl.ds(base + q * rq, rq)], sem_out))
    for h in out_copies:
        h.wait()


@functools.lru_cache(maxsize=None)
def _make_sc_gate(n_rows):
    rpw = n_rows // NW
    return pl.kernel(
        functools.partial(_sc_gate_body, rpw),
        out_type=[
            jax.ShapeDtypeStruct((n_rows, K), jnp.float32),
            jax.ShapeDtypeStruct((n_rows, K), jnp.int32),
        ],
        mesh=plsc.VectorSubcoreMesh(core_axis_name="c", subcore_axis_name="s"),
        compiler_params=pltpu.CompilerParams(needs_layout_passes=False),
        scratch_types=[
            pltpu.VMEM((rpw, N_EXPERTS), jnp.float32),
            pltpu.VMEM((rpw, K), jnp.float32),
            pltpu.VMEM((rpw, K), jnp.int32),
            [pltpu.SemaphoreType.DMA] * N_Q,
            pltpu.SemaphoreType.DMA,
        ],
    )


@jax.jit
def kernel(x, weight, bias):
    scores = _tc_scores(x, weight, bias.reshape(1, N_EXPERTS))
    return tuple(_make_sc_gate(x.shape[0])(scores))
